# baseline (speedup 1.0000x reference)
import functools

import jax
import jax.numpy as jnp
from jax import lax
from jax.experimental import pallas as pl
from jax.experimental.pallas import tpu as pltpu
from jax.experimental.pallas import tpu_sc as plsc

_LANES = 16
_GATHER_CHUNK = 128


def kernel(src, rel, dst, entity_embedding, relation_embedding):
    batch = src.shape[0]
    hidden = entity_embedding.shape[1]
    info = plsc.get_sparse_core_info()
    num_cores, num_subcores = info.num_cores, info.num_subcores
    num_workers = num_cores * num_subcores
    per_w = batch // num_workers
    n_chunks = per_w // _GATHER_CHUNK

    mesh = plsc.VectorSubcoreMesh(core_axis_name="c", subcore_axis_name="s")

    @functools.partial(
        pl.kernel,
        mesh=mesh,
        out_type=jax.ShapeDtypeStruct((batch,), jnp.float32),
        compiler_params=pltpu.CompilerParams(
            needs_layout_passes=False, use_tc_tiling_on_sc=False),
        scratch_types=[
            pltpu.VMEM((per_w,), jnp.int32),
            pltpu.VMEM((per_w,), jnp.int32),
            pltpu.VMEM((per_w,), jnp.int32),
            pltpu.VMEM((per_w, hidden), jnp.float32),
            pltpu.VMEM((per_w, hidden), jnp.float32),
            pltpu.VMEM((per_w, hidden), jnp.float32),
            pltpu.VMEM((per_w,), jnp.float32),
            pltpu.SemaphoreType.DMA,
        ],
    )
    def distmult(src_h, rel_h, dst_h, ent_h, relemb_h, out_h,
                 si_v, ri_v, di_v, sr_v, rr_v, dr_v, o_v, sem):
        wid = lax.axis_index("s") * num_cores + lax.axis_index("c")
        base = wid * per_w

        pltpu.sync_copy(src_h.at[pl.ds(base, per_w)], si_v)
        pltpu.sync_copy(rel_h.at[pl.ds(base, per_w)], ri_v)
        pltpu.sync_copy(dst_h.at[pl.ds(base, per_w)], di_v)

        copies = []
        for j in range(n_chunks):
            sl = pl.ds(j * _GATHER_CHUNK, _GATHER_CHUNK)
            copies.append(pltpu.async_copy(ent_h.at[si_v.at[sl]], sr_v.at[sl], sem))
            copies.append(pltpu.async_copy(relemb_h.at[ri_v.at[sl]], rr_v.at[sl], sem))
            copies.append(pltpu.async_copy(ent_h.at[di_v.at[sl]], dr_v.at[sl], sem))
        for c in copies:
            c.wait()

        def body(blk, carry):
            rows = blk * _LANES + lax.iota(jnp.int32, _LANES)
            acc = jnp.zeros((_LANES,), jnp.float32)
            for d in range(hidden):
                dv = jnp.full((_LANES,), d, jnp.int32)
                s_ = plsc.load_gather(sr_v, [rows, dv])
                r_ = plsc.load_gather(rr_v, [rows, dv])
                t_ = plsc.load_gather(dr_v, [rows, dv])
                acc = acc + s_ * r_ * t_
            o_v[pl.ds(blk * _LANES, _LANES)] = acc
            return carry

        lax.fori_loop(0, per_w // _LANES, body, 0)
        pltpu.sync_copy(o_v, out_h.at[pl.ds(base, per_w)])

    return distmult(src, rel, dst, entity_embedding, relation_embedding)


# speedup vs baseline: 1.5880x; 1.5880x over previous
import functools

import jax
import jax.numpy as jnp
from jax import lax
from jax.experimental import pallas as pl
from jax.experimental.pallas import tpu as pltpu
from jax.experimental.pallas import tpu_sc as plsc

_LANES = 16
_CHUNKS = 2


def kernel(src, rel, dst, entity_embedding, relation_embedding):
    batch = src.shape[0]
    hidden = entity_embedding.shape[1]
    info = plsc.get_sparse_core_info()
    num_cores, num_subcores = info.num_cores, info.num_subcores
    num_workers = num_cores * num_subcores
    per_w = batch // num_workers
    chunk = per_w // _CHUNKS

    mesh = plsc.VectorSubcoreMesh(core_axis_name="c", subcore_axis_name="s")

    @functools.partial(
        pl.kernel,
        mesh=mesh,
        out_type=jax.ShapeDtypeStruct((batch,), jnp.float32),
        compiler_params=pltpu.CompilerParams(needs_layout_passes=False),
        scratch_types=[
            pltpu.VMEM((per_w,), jnp.int32),
            pltpu.VMEM((per_w,), jnp.int32),
            pltpu.VMEM((per_w,), jnp.int32),
            pltpu.VMEM((chunk, hidden), jnp.float32),
            pltpu.VMEM((chunk, hidden), jnp.float32),
            pltpu.VMEM((chunk, hidden), jnp.float32),
            pltpu.VMEM((per_w,), jnp.float32),
            pltpu.SemaphoreType.DMA,
        ],
    )
    def distmult(src_h, rel_h, dst_h, ent_h, relemb_h, out_h,
                 si_v, ri_v, di_v, sr_v, rr_v, dr_v, o_v, sem):
        wid = lax.axis_index("s") * num_cores + lax.axis_index("c")
        base = wid * per_w

        pltpu.sync_copy(src_h.at[pl.ds(base, per_w)], si_v)
        pltpu.sync_copy(rel_h.at[pl.ds(base, per_w)], ri_v)
        pltpu.sync_copy(dst_h.at[pl.ds(base, per_w)], di_v)

        def issue_chunk(idx_v, table_h, rows_v, c):
            def issue(blk, carry):
                vec = idx_v[pl.ds(c * chunk + blk * _LANES, _LANES)]
                for j in range(_LANES):
                    row = vec[j]
                    pltpu.async_copy(
                        table_h.at[pl.ds(row, 1), :],
                        rows_v.at[pl.ds(blk * _LANES + j, 1), :],
                        sem)
                return carry

            lax.fori_loop(0, chunk // _LANES, issue, 0)

        for c in range(_CHUNKS):
            issue_chunk(si_v, ent_h, sr_v, c)
            issue_chunk(ri_v, relemb_h, rr_v, c)
            issue_chunk(di_v, ent_h, dr_v, c)
            for rows_v in (sr_v, rr_v, dr_v):
                pltpu.make_async_copy(
                    ent_h.at[pl.ds(0, chunk), :], rows_v, sem).wait()

            def body(blk, carry):
                rows = blk * _LANES + lax.iota(jnp.int32, _LANES)
                acc = jnp.zeros((_LANES,), jnp.float32)
                for d in range(hidden):
                    dv = jnp.full((_LANES,), d, jnp.int32)
                    s_ = plsc.load_gather(sr_v, [rows, dv])
                    r_ = plsc.load_gather(rr_v, [rows, dv])
                    t_ = plsc.load_gather(dr_v, [rows, dv])
                    acc = acc + s_ * r_ * t_
                o_v[pl.ds(c * chunk + blk * _LANES, _LANES)] = acc
                return carry

            lax.fori_loop(0, chunk // _LANES, body, 0)

        pltpu.sync_copy(o_v, out_h.at[pl.ds(base, per_w)])

    return distmult(src, rel, dst, entity_embedding, relation_embedding)
